# baseline (speedup 1.0000x reference)
import functools
import math

import jax
import jax.numpy as jnp
from jax import lax
from jax.experimental import pallas as pl
from jax.experimental.pallas import tpu as pltpu
from jax.experimental.pallas import tpu_sc as plsc

_PHI = (1 + math.sqrt(5)) / 2
_ALPHA = 1 / _PHI
_BETA = 1 / _PHI ** 2

_DIM = 1910
_NPAD = 2048
_KCOLS = 955
_H_ELEMS = _KCOLS * _NPAD

_CB = 2048
_NSUB = 16

_ZB = 10240
_OUT_CHUNKS = _H_ELEMS // _ZB

_BM = 512


def _make_scatter_body(nz):
    ncht = -(-nz // _CB)
    nh = -(-ncht // _NSUB)

    def body(widx_hbm, v_hbm, rct_hbm, vt_hbm, z_hbm, out_hbm,
             shared, sem):
        cid = lax.axis_index("c")
        sid = lax.axis_index("s")
        base = cid * _KCOLS

        def inner(rc_v, v_v, idx_b):
            def zchunk(it, _):
                ch = sid + it * _NSUB
                @pl.when(ch < _OUT_CHUNKS)
                def _():
                    pltpu.async_copy(z_hbm, shared.at[pl.ds(ch * _ZB, _ZB)],
                                     sem)
                return 0
            lax.fori_loop(0, -(-_OUT_CHUNKS // _NSUB), zchunk, 0)

            def zdrain(it, _):
                ch = sid + it * _NSUB
                @pl.when(ch < _OUT_CHUNKS)
                def _():
                    pltpu.make_async_copy(
                        z_hbm, shared.at[pl.ds(ch * _ZB, _ZB)], sem).wait()
                return 0
            lax.fori_loop(0, -(-_OUT_CHUNKS // _NSUB), zdrain, 0)
            plsc.subcore_barrier()

            def chunk(h, _):
                gch = sid + h * _NSUB

                @pl.when(gch < ncht)
                def _():
                    last = gch == ncht - 1

                    @pl.when(last)
                    def _():
                        pltpu.async_copy(rct_hbm, rc_v, sem)
                        pltpu.async_copy(vt_hbm, v_v, sem)

                    @pl.when(jnp.logical_not(last))
                    def _():
                        start = gch * _CB
                        pltpu.async_copy(widx_hbm.at[:, pl.ds(start, _CB)],
                                         rc_v, sem)
                        pltpu.async_copy(v_hbm.at[pl.ds(start, _CB)], v_v,
                                         sem)
                    pltpu.make_async_copy(rct_hbm, rc_v, sem).wait()
                    pltpu.make_async_copy(vt_hbm, v_v, sem).wait()

                    def row(j, _):
                        for k in range(8):
                            off = j * 128 + k * 16
                            r16 = rc_v[0, pl.ds(off, 16)]
                            c16 = rc_v[1, pl.ds(off, 16)]
                            loc = c16 - base
                            flat = loc * _NPAD + r16
                            ok = plsc.bitcast(loc, jnp.uint32) < _KCOLS
                            idx_b[j, pl.ds(k * 16, 16)] = jnp.where(
                                ok, flat, -1)
                        return 0
                    lax.fori_loop(0, _CB // 128, row, 0)
                    descs = [
                        pltpu.async_copy(
                            v_v.at[pl.ds(j * 128, 128)],
                            shared.at[plsc.Indices(idx_b.at[j],
                                                   ignored_value=-1)],
                            sem, add=True)
                        for j in range(_CB // 128)
                    ]
                    for d in descs:
                        d.wait()
                return 0
            lax.fori_loop(0, nh, chunk, 0)
            plsc.subcore_barrier()

            def ochunk(it, _):
                ch = sid + it * _NSUB
                @pl.when(ch < _OUT_CHUNKS)
                def _():
                    pltpu.async_copy(
                        shared.at[pl.ds(ch * _ZB, _ZB)],
                        out_hbm.at[pl.ds(cid * _H_ELEMS + ch * _ZB, _ZB)],
                        sem)
                return 0
            lax.fori_loop(0, -(-_OUT_CHUNKS // _NSUB), ochunk, 0)

            def odrain(it, _):
                ch = sid + it * _NSUB
                @pl.when(ch < _OUT_CHUNKS)
                def _():
                    pltpu.make_async_copy(
                        shared.at[pl.ds(ch * _ZB, _ZB)],
                        out_hbm.at[pl.ds(cid * _H_ELEMS + ch * _ZB, _ZB)],
                        sem).wait()
                return 0
            lax.fori_loop(0, -(-_OUT_CHUNKS // _NSUB), odrain, 0)

        pl.run_scoped(
            inner,
            pltpu.VMEM((2, _CB), jnp.int32),
            pltpu.VMEM((_CB,), jnp.float32),
            pltpu.VMEM((_CB // 128, 128), jnp.int32),
        )

    return body


def _sc_scatter(W_indices, W_values):
    nz = W_values.shape[0]
    ncht = -(-nz // _CB)
    s0 = (ncht - 1) * _CB
    tail = nz - s0
    rct = jnp.zeros((2, _CB), jnp.int32).at[:, :tail].set(W_indices[:, s0:])
    vt = jnp.zeros((_CB,), jnp.float32).at[:tail].set(W_values[s0:])
    z = jnp.zeros((_ZB,), jnp.float32)
    mesh = plsc.VectorSubcoreMesh(core_axis_name="c", subcore_axis_name="s")
    f = functools.partial(
        pl.kernel,
        out_type=jax.ShapeDtypeStruct((2 * _H_ELEMS,), jnp.float32),
        mesh=mesh,
        scratch_types=[
            pltpu.VMEM_SHARED((_H_ELEMS,), jnp.float32),
            pltpu.SemaphoreType.DMA,
        ],
    )(_make_scatter_body(nz))
    return f(W_indices, W_values, rct, vt, z)


def _ffn_body(x_ref, w_ref, o_ref):
    xf = x_ref[0]
    wx = lax.dot_general(
        xf.astype(jnp.bfloat16), w_ref[...], (((1,), (0,)), ((), ())),
        preferred_element_type=jnp.float32,
    )
    o_ref[0] = _ALPHA * xf + _BETA * wx[:, : o_ref.shape[2]]


def _ffn_matmul(x, Wp):
    B, S, dim = x.shape
    return pl.pallas_call(
        _ffn_body,
        grid=(B, S // _BM),
        in_specs=[
            pl.BlockSpec((1, _BM, dim), lambda b, i: (b, i, 0)),
            pl.BlockSpec(Wp.shape, lambda b, i: (0, 0)),
        ],
        out_specs=pl.BlockSpec((1, _BM, dim), lambda b, i: (b, i, 0)),
        out_shape=jax.ShapeDtypeStruct((B, S, dim), jnp.float32),
    )(x, Wp)


def kernel(x, W_indices, W_values):
    dim = x.shape[-1]
    Wp = _sc_scatter(W_indices, W_values)
    Wp = Wp.reshape(dim, _NPAD).astype(jnp.bfloat16)
    return _ffn_matmul(x, Wp)


# speedup vs baseline: 1.0644x; 1.0644x over previous
import functools
import math

import jax
import jax.numpy as jnp
from jax import lax
from jax.experimental import pallas as pl
from jax.experimental.pallas import tpu as pltpu
from jax.experimental.pallas import tpu_sc as plsc

_PHI = (1 + math.sqrt(5)) / 2
_ALPHA = 1 / _PHI
_BETA = 1 / _PHI ** 2

_DIM = 1910
_NPAD = 2048
_KCOLS = 955
_H_ELEMS = _KCOLS * _NPAD

_CB = 2048
_NSUB = 16

_ZB = 10240
_OUT_CHUNKS = _H_ELEMS // _ZB

_BM = 1024


def _make_scatter_body(nz):
    ncht = -(-nz // _CB)
    nh = -(-ncht // _NSUB)

    def body(widx_hbm, v_hbm, rct_hbm, vt_hbm, z_hbm, out_hbm,
             shared, sem):
        cid = lax.axis_index("c")
        sid = lax.axis_index("s")
        base = cid * _KCOLS

        def inner(rc_v, v_v, idx_b):
            def zchunk(it, _):
                ch = sid + it * _NSUB
                @pl.when(ch < _OUT_CHUNKS)
                def _():
                    pltpu.async_copy(z_hbm, shared.at[pl.ds(ch * _ZB, _ZB)],
                                     sem)
                return 0
            lax.fori_loop(0, -(-_OUT_CHUNKS // _NSUB), zchunk, 0)

            def zdrain(it, _):
                ch = sid + it * _NSUB
                @pl.when(ch < _OUT_CHUNKS)
                def _():
                    pltpu.make_async_copy(
                        z_hbm, shared.at[pl.ds(ch * _ZB, _ZB)], sem).wait()
                return 0
            lax.fori_loop(0, -(-_OUT_CHUNKS // _NSUB), zdrain, 0)
            plsc.subcore_barrier()

            def chunk(h, _):
                gch = sid + h * _NSUB

                @pl.when(gch < ncht)
                def _():
                    last = gch == ncht - 1

                    @pl.when(last)
                    def _():
                        pltpu.async_copy(rct_hbm, rc_v, sem)
                        pltpu.async_copy(vt_hbm, v_v, sem)

                    @pl.when(jnp.logical_not(last))
                    def _():
                        start = gch * _CB
                        pltpu.async_copy(widx_hbm.at[:, pl.ds(start, _CB)],
                                         rc_v, sem)
                        pltpu.async_copy(v_hbm.at[pl.ds(start, _CB)], v_v,
                                         sem)
                    pltpu.make_async_copy(rct_hbm, rc_v, sem).wait()
                    pltpu.make_async_copy(vt_hbm, v_v, sem).wait()

                    def row(j, _):
                        for k in range(8):
                            off = j * 128 + k * 16
                            r16 = rc_v[0, pl.ds(off, 16)]
                            c16 = rc_v[1, pl.ds(off, 16)]
                            loc = c16 - base
                            flat = loc * _NPAD + r16
                            ok = plsc.bitcast(loc, jnp.uint32) < _KCOLS
                            idx_b[j, pl.ds(k * 16, 16)] = jnp.where(
                                ok, flat, -1)
                        return 0
                    lax.fori_loop(0, _CB // 128, row, 0)
                    descs = [
                        pltpu.async_copy(
                            v_v.at[pl.ds(j * 128, 128)],
                            shared.at[plsc.Indices(idx_b.at[j],
                                                   ignored_value=-1)],
                            sem, add=True)
                        for j in range(_CB // 128)
                    ]
                    for d in descs:
                        d.wait()
                return 0
            lax.fori_loop(0, nh, chunk, 0)
            plsc.subcore_barrier()

            def ochunk(it, _):
                ch = sid + it * _NSUB
                @pl.when(ch < _OUT_CHUNKS)
                def _():
                    pltpu.async_copy(
                        shared.at[pl.ds(ch * _ZB, _ZB)],
                        out_hbm.at[pl.ds(cid * _H_ELEMS + ch * _ZB, _ZB)],
                        sem)
                return 0
            lax.fori_loop(0, -(-_OUT_CHUNKS // _NSUB), ochunk, 0)

            def odrain(it, _):
                ch = sid + it * _NSUB
                @pl.when(ch < _OUT_CHUNKS)
                def _():
                    pltpu.make_async_copy(
                        shared.at[pl.ds(ch * _ZB, _ZB)],
                        out_hbm.at[pl.ds(cid * _H_ELEMS + ch * _ZB, _ZB)],
                        sem).wait()
                return 0
            lax.fori_loop(0, -(-_OUT_CHUNKS // _NSUB), odrain, 0)

        pl.run_scoped(
            inner,
            pltpu.VMEM((2, _CB), jnp.int32),
            pltpu.VMEM((_CB,), jnp.float32),
            pltpu.VMEM((_CB // 128, 128), jnp.int32),
        )

    return body


def _sc_scatter(W_indices, W_values):
    nz = W_values.shape[0]
    ncht = -(-nz // _CB)
    s0 = (ncht - 1) * _CB
    tail = nz - s0
    rct = jnp.zeros((2, _CB), jnp.int32).at[:, :tail].set(W_indices[:, s0:])
    vt = jnp.zeros((_CB,), jnp.float32).at[:tail].set(W_values[s0:])
    z = jnp.zeros((_ZB,), jnp.float32)
    mesh = plsc.VectorSubcoreMesh(core_axis_name="c", subcore_axis_name="s")
    f = functools.partial(
        pl.kernel,
        out_type=jax.ShapeDtypeStruct((2 * _H_ELEMS,), jnp.float32),
        mesh=mesh,
        scratch_types=[
            pltpu.VMEM_SHARED((_H_ELEMS,), jnp.float32),
            pltpu.SemaphoreType.DMA,
        ],
    )(_make_scatter_body(nz))
    return f(W_indices, W_values, rct, vt, z)


def _ffn_body(x_ref, w_ref, o_ref):
    xf = x_ref[...]
    wx = lax.dot_general(
        xf.astype(jnp.bfloat16), w_ref[...], (((1,), (0,)), ((), ())),
        preferred_element_type=jnp.float32,
    )
    o_ref[...] = _ALPHA * xf + _BETA * wx[:, : o_ref.shape[1]]


def _ffn_matmul(xf, Wp):
    M, dim = xf.shape
    return pl.pallas_call(
        _ffn_body,
        grid=(M // _BM,),
        in_specs=[
            pl.BlockSpec((_BM, dim), lambda i: (i, 0)),
            pl.BlockSpec(Wp.shape, lambda i: (0, 0)),
        ],
        out_specs=pl.BlockSpec((_BM, dim), lambda i: (i, 0)),
        out_shape=jax.ShapeDtypeStruct((M, dim), jnp.float32),
    )(xf, Wp)


def kernel(x, W_indices, W_values):
    dim = x.shape[-1]
    xf = x.reshape(-1, dim)
    Wp = _sc_scatter(W_indices, W_values)
    Wp = Wp.reshape(dim, _NPAD).astype(jnp.bfloat16)
    out = _ffn_matmul(xf, Wp)
    return out.reshape(x.shape)


# speedup vs baseline: 1.1270x; 1.0588x over previous
import functools
import math

import jax
import jax.numpy as jnp
from jax import lax
from jax.experimental import pallas as pl
from jax.experimental.pallas import tpu as pltpu
from jax.experimental.pallas import tpu_sc as plsc

_PHI = (1 + math.sqrt(5)) / 2
_ALPHA = 1 / _PHI
_BETA = 1 / _PHI ** 2

_DIM = 1910
_NPAD = 2048
_KCOLS = 955
_H_ELEMS = _KCOLS * _NPAD

_CB = 2048
_NSUB = 16

_ZB = 10240
_OUT_CHUNKS = _H_ELEMS // _ZB

_BM = 512


def _make_scatter_body(nz):
    ncht = -(-nz // _CB)
    nh = -(-ncht // _NSUB)

    def body(widx_hbm, v_hbm, rct_hbm, vt_hbm, z_hbm, out_hbm,
             shared, sem, sem_rc, sem_v, sem_sc):
        cid = lax.axis_index("c")
        sid = lax.axis_index("s")
        base = cid * _KCOLS

        def inner(rc_v, v_v, idx_b):
            def zchunk(it, _):
                ch = sid + it * _NSUB
                @pl.when(ch < _OUT_CHUNKS)
                def _():
                    pltpu.async_copy(z_hbm, shared.at[pl.ds(ch * _ZB, _ZB)],
                                     sem)
                return 0
            lax.fori_loop(0, -(-_OUT_CHUNKS // _NSUB), zchunk, 0)

            def zdrain(it, _):
                ch = sid + it * _NSUB
                @pl.when(ch < _OUT_CHUNKS)
                def _():
                    pltpu.make_async_copy(
                        z_hbm, shared.at[pl.ds(ch * _ZB, _ZB)], sem).wait()
                return 0
            lax.fori_loop(0, -(-_OUT_CHUNKS // _NSUB), zdrain, 0)
            plsc.subcore_barrier()

            def fire_rc(h):
                gch = sid + h * _NSUB
                @pl.when(gch < ncht)
                def _():
                    @pl.when(gch == ncht - 1)
                    def _():
                        pltpu.async_copy(rct_hbm, rc_v, sem_rc)
                    @pl.when(gch < ncht - 1)
                    def _():
                        pltpu.async_copy(
                            widx_hbm.at[:, pl.ds(gch * _CB, _CB)], rc_v,
                            sem_rc)

            def fire_v(h):
                gch = sid + h * _NSUB
                @pl.when(gch < ncht)
                def _():
                    @pl.when(gch == ncht - 1)
                    def _():
                        pltpu.async_copy(vt_hbm, v_v, sem_v)
                    @pl.when(gch < ncht - 1)
                    def _():
                        pltpu.async_copy(v_hbm.at[pl.ds(gch * _CB, _CB)],
                                         v_v, sem_v)

            fire_rc(0)
            fire_v(0)

            def chunk(h, _):
                gch = sid + h * _NSUB

                @pl.when(gch < ncht)
                def _():
                    pltpu.make_async_copy(rct_hbm, rc_v, sem_rc).wait()
                    pltpu.make_async_copy(vt_hbm, v_v, sem_v).wait()

                    def row(j, _):
                        for k in range(8):
                            off = j * 128 + k * 16
                            r16 = rc_v[0, pl.ds(off, 16)]
                            c16 = rc_v[1, pl.ds(off, 16)]
                            loc = c16 - base
                            flat = loc * _NPAD + r16
                            ok = plsc.bitcast(loc, jnp.uint32) < _KCOLS
                            idx_b[j, pl.ds(k * 16, 16)] = jnp.where(
                                ok, flat, -1)
                        return 0
                    lax.fori_loop(0, _CB // 128, row, 0)
                    descs = [
                        pltpu.async_copy(
                            v_v.at[pl.ds(j * 128, 128)],
                            shared.at[plsc.Indices(idx_b.at[j],
                                                   ignored_value=-1)],
                            sem_sc, add=True)
                        for j in range(_CB // 128)
                    ]
                    fire_rc(h + 1)
                    for d in descs:
                        d.wait()
                    fire_v(h + 1)
                return 0
            lax.fori_loop(0, nh, chunk, 0)
            plsc.subcore_barrier()

            def ochunk(it, _):
                ch = sid + it * _NSUB
                @pl.when(ch < _OUT_CHUNKS)
                def _():
                    pltpu.async_copy(
                        shared.at[pl.ds(ch * _ZB, _ZB)],
                        out_hbm.at[pl.ds(cid * _H_ELEMS + ch * _ZB, _ZB)],
                        sem)
                return 0
            lax.fori_loop(0, -(-_OUT_CHUNKS // _NSUB), ochunk, 0)

            def odrain(it, _):
                ch = sid + it * _NSUB
                @pl.when(ch < _OUT_CHUNKS)
                def _():
                    pltpu.make_async_copy(
                        shared.at[pl.ds(ch * _ZB, _ZB)],
                        out_hbm.at[pl.ds(cid * _H_ELEMS + ch * _ZB, _ZB)],
                        sem).wait()
                return 0
            lax.fori_loop(0, -(-_OUT_CHUNKS // _NSUB), odrain, 0)

        pl.run_scoped(
            inner,
            pltpu.VMEM((2, _CB), jnp.int32),
            pltpu.VMEM((_CB,), jnp.float32),
            pltpu.VMEM((_CB // 128, 128), jnp.int32),
        )

    return body


def _sc_scatter(W_indices, W_values):
    nz = W_values.shape[0]
    ncht = -(-nz // _CB)
    s0 = (ncht - 1) * _CB
    tail = nz - s0
    rct = jnp.zeros((2, _CB), jnp.int32).at[:, :tail].set(W_indices[:, s0:])
    vt = jnp.zeros((_CB,), jnp.float32).at[:tail].set(W_values[s0:])
    z = jnp.zeros((_ZB,), jnp.float32)
    mesh = plsc.VectorSubcoreMesh(core_axis_name="c", subcore_axis_name="s")
    f = functools.partial(
        pl.kernel,
        out_type=jax.ShapeDtypeStruct((2 * _H_ELEMS,), jnp.float32),
        mesh=mesh,
        scratch_types=[
            pltpu.VMEM_SHARED((_H_ELEMS,), jnp.float32),
            pltpu.SemaphoreType.DMA,
            pltpu.SemaphoreType.DMA,
            pltpu.SemaphoreType.DMA,
            pltpu.SemaphoreType.DMA,
        ],
    )(_make_scatter_body(nz))
    return f(W_indices, W_values, rct, vt, z)


def _ffn_body(x_ref, w_ref, o_ref):
    xf = x_ref[...]
    wx = lax.dot_general(
        xf.astype(jnp.bfloat16), w_ref[...], (((1,), (0,)), ((), ())),
        preferred_element_type=jnp.float32,
    )
    o_ref[...] = _ALPHA * xf + _BETA * wx[:, : o_ref.shape[1]]


def _ffn_matmul(xf, Wp):
    M, dim = xf.shape
    return pl.pallas_call(
        _ffn_body,
        grid=(M // _BM,),
        in_specs=[
            pl.BlockSpec((_BM, dim), lambda i: (i, 0)),
            pl.BlockSpec(Wp.shape, lambda i: (0, 0)),
        ],
        out_specs=pl.BlockSpec((_BM, dim), lambda i: (i, 0)),
        out_shape=jax.ShapeDtypeStruct((M, dim), jnp.float32),
    )(xf, Wp)


def kernel(x, W_indices, W_values):
    dim = x.shape[-1]
    xf = x.reshape(-1, dim)
    Wp = _sc_scatter(W_indices, W_values)
    Wp = Wp.reshape(dim, _NPAD).astype(jnp.bfloat16)
    out = _ffn_matmul(xf, Wp)
    return out.reshape(x.shape)
